# Initial kernel scaffold; baseline (speedup 1.0000x reference)
import jax
import jax.numpy as jnp
from jax.experimental import pallas as pl


def kernel(x_member, x_provider, edge_index, W1_l, b1_l, W1_r, W2_l, b2_l, W2_r, Wd, bd):
    raise NotImplementedError("write your pallas kernel here")


# speedup vs baseline: 4.7513x; 4.7513x over previous
import functools

import jax
import jax.numpy as jnp
from jax import lax
from jax.experimental import pallas as pl
from jax.experimental.pallas import tpu as pltpu
from jax.experimental.pallas import tpu_sc as plsc

F32 = jnp.float32

NC = 2
NS = 16
NW = NC * NS
ECH = 80
RCH = 128


def _mesh():
    return plsc.VectorSubcoreMesh(core_axis_name="c", subcore_axis_name="s",
                                  num_cores=NC, num_subcores=NS)


def _mm_kernel(x_ref, w_ref, a_out, b_out, split):
    o = jnp.dot(x_ref[:], w_ref[:], preferred_element_type=F32)
    a_out[:] = o[:, :split]
    b_out[:] = o[:, split:]


def _tc_linear_pair(x, wcat, split, blk=1000):
    r, k = x.shape
    m = wcat.shape[1]
    return pl.pallas_call(
        functools.partial(_mm_kernel, split=split),
        grid=(r // blk,),
        in_specs=[
            pl.BlockSpec((blk, k), lambda i: (i, 0)),
            pl.BlockSpec((k, m), lambda i: (0, 0)),
        ],
        out_specs=[
            pl.BlockSpec((blk, split), lambda i: (i, 0)),
            pl.BlockSpec((blk, m - split), lambda i: (i, 0)),
        ],
        out_shape=[
            jax.ShapeDtypeStruct((r, split), F32),
            jax.ShapeDtypeStruct((r, m - split), F32),
        ],
    )(x, wcat)


def _combine2_kernel(agg_ref, cnt_ref, xr_ref, b_ref, w_ref, a_out, b_out,
                     split, relu, pad_to):
    inv = 1.0 / jnp.maximum(cnt_ref[:, 0:1], 1.0)
    h = agg_ref[:] * inv + b_ref[:] + xr_ref[:]
    if relu:
        h = jnp.maximum(h, 0.0)
    o = jnp.dot(h, w_ref[:], preferred_element_type=F32)
    a = o[:, :split]
    if pad_to > split:
        a = jnp.concatenate(
            [a, jnp.zeros((a.shape[0], pad_to - split), F32)], axis=1)
    a_out[:] = a
    b_out[:] = o[:, split:]


def _tc_combine_linear(agg, cntf, xr, bvec, wcat, split, relu, pad_to,
                       blk=1000):
    r, d = agg.shape
    dh = xr.shape[1]
    m = wcat.shape[1]
    return pl.pallas_call(
        functools.partial(_combine2_kernel, split=split, relu=relu,
                          pad_to=pad_to),
        grid=(r // blk,),
        in_specs=[
            pl.BlockSpec((blk, d), lambda i: (i, 0)),
            pl.BlockSpec((blk, 128), lambda i: (i, 0)),
            pl.BlockSpec((blk, dh), lambda i: (i, 0)),
            pl.BlockSpec((1, dh), lambda i: (0, 0)),
            pl.BlockSpec((dh, m), lambda i: (0, 0)),
        ],
        out_specs=[
            pl.BlockSpec((blk, pad_to), lambda i: (i, 0)),
            pl.BlockSpec((blk, m - split), lambda i: (i, 0)),
        ],
        out_shape=[
            jax.ShapeDtypeStruct((r, pad_to), F32),
            jax.ShapeDtypeStruct((r, m - split), F32),
        ],
    )(agg, cntf, xr, bvec, wcat)


def _final_kernel(agg_ref, cnt_ref, hr_ref, b2_ref, wd_ref, bd_ref,
                  z_out, xh_out, d, pad_to):
    inv = 1.0 / jnp.maximum(cnt_ref[:, 0:1], 1.0)
    z = agg_ref[:, :d] * inv + b2_ref[:] + hr_ref[:]
    zp = jnp.concatenate([z, jnp.zeros((z.shape[0], pad_to - d), F32)], axis=1)
    z_out[:] = zp
    xh_out[:] = jnp.dot(z, wd_ref[:], preferred_element_type=F32) + bd_ref[:]


def _tc_final(agg2, cntf, hr2, b2, wdt, bd, pad_to, blk=1000):
    r, da = agg2.shape
    d = hr2.shape[1]
    m = wdt.shape[1]
    return pl.pallas_call(
        functools.partial(_final_kernel, d=d, pad_to=pad_to),
        grid=(r // blk,),
        in_specs=[
            pl.BlockSpec((blk, da), lambda i: (i, 0)),
            pl.BlockSpec((blk, 128), lambda i: (i, 0)),
            pl.BlockSpec((blk, d), lambda i: (i, 0)),
            pl.BlockSpec((1, d), lambda i: (0, 0)),
            pl.BlockSpec((d, m), lambda i: (0, 0)),
            pl.BlockSpec((1, m), lambda i: (0, 0)),
        ],
        out_specs=[
            pl.BlockSpec((blk, pad_to), lambda i: (i, 0)),
            pl.BlockSpec((blk, m), lambda i: (i, 0)),
        ],
        out_shape=[
            jax.ShapeDtypeStruct((r, pad_to), F32),
            jax.ShapeDtypeStruct((r, m), F32),
        ],
    )(agg2, cntf, hr2, b2, wdt, bd)


def _zero_acc(s, rows_v, acc, n_half):
    nz = (n_half + RCH * NS - 1) // (RCH * NS)
    maxoff = n_half - RCH

    def zbody(k, carry):
        off = jnp.minimum((k * NS + s) * RCH, maxoff)
        pltpu.sync_copy(rows_v, acc.at[pl.ds(off, RCH)])
        return carry

    lax.fori_loop(0, nz, zbody, 0)


def _flush_acc(c, s, stage_v, acc, out, n_half):
    nz = (n_half + RCH * NS - 1) // (RCH * NS)
    maxoff = n_half - RCH

    def fbody(k, carry):
        off = jnp.minimum((k * NS + s) * RCH, maxoff)
        pltpu.sync_copy(acc.at[pl.ds(off, RCH)], stage_v)
        pltpu.sync_copy(stage_v, out.at[pl.ds(c * n_half + off, RCH)])
        return carry

    lax.fori_loop(0, nz, fbody, 0)


def _make_sc_agg(n_half, d, e2):
    e_half = e2 // 2
    per_w = e_half // NS
    assert per_w % ECH == 0
    nch = per_w // ECH

    @functools.partial(
        pl.kernel, mesh=_mesh(),
        out_type=jax.ShapeDtypeStruct((2 * n_half, d), F32),
        scratch_types=[
            pltpu.VMEM((ECH,), jnp.int32),
            pltpu.VMEM((ECH,), jnp.int32),
            pltpu.VMEM((ECH, d), F32),
            pltpu.VMEM((RCH, d), F32),
            pltpu.VMEM_SHARED((n_half, d), F32),
        ])
    def sc_agg(table, gidx, sidx, zrow, agg_out,
               src_v, dst_v, rows_v, stage_v, acc):
        c = lax.axis_index("c")
        s = lax.axis_index("s")
        wbase = (c * NS + s) * per_w

        pltpu.sync_copy(zrow, stage_v)
        _zero_acc(s, stage_v, acc, n_half)
        plsc.subcore_barrier()

        def body(k, carry):
            b = wbase + k * ECH
            pltpu.sync_copy(gidx.at[pl.ds(b, ECH)], src_v)
            pltpu.sync_copy(sidx.at[pl.ds(b, ECH)], dst_v)
            pltpu.sync_copy(table.at[src_v], rows_v)
            pltpu.sync_copy(rows_v, acc.at[dst_v], add=True)
            return carry

        lax.fori_loop(0, nch, body, 0)
        plsc.subcore_barrier()
        _flush_acc(c, s, stage_v, acc, agg_out, n_half)

    return sc_agg


def _make_sc_cnt(n_half, e2):
    e_half = e2 // 2
    per_w = e_half // NS
    assert per_w % ECH == 0
    nch = per_w // ECH

    @functools.partial(
        pl.kernel, mesh=_mesh(),
        out_type=jax.ShapeDtypeStruct((2 * n_half, 128), F32),
        scratch_types=[
            pltpu.VMEM((ECH,), jnp.int32),
            pltpu.VMEM((ECH, 128), F32),
            pltpu.VMEM((RCH, 128), F32),
            pltpu.VMEM_SHARED((n_half, 128), F32),
        ])
    def sc_cnt(sidx, zrow, onerow, cnt_out, dst_v, ones_v, stage_v, acc):
        c = lax.axis_index("c")
        s = lax.axis_index("s")
        wbase = (c * NS + s) * per_w

        pltpu.sync_copy(zrow, stage_v)
        pltpu.sync_copy(onerow, ones_v)
        _zero_acc(s, stage_v, acc, n_half)
        plsc.subcore_barrier()

        def body(k, carry):
            b = wbase + k * ECH
            pltpu.sync_copy(sidx.at[pl.ds(b, ECH)], dst_v)
            pltpu.sync_copy(ones_v, acc.at[dst_v], add=True)
            return carry

        lax.fori_loop(0, nch, body, 0)
        plsc.subcore_barrier()
        _flush_acc(c, s, stage_v, acc, cnt_out, n_half)

    return sc_cnt


def _make_sc_edge_logits(e, dtab, d):
    per_w = e // NW
    assert per_w % ECH == 0
    nch = per_w // ECH
    assert d % 16 == 0
    nseg = d // 16

    @functools.partial(
        pl.kernel, mesh=_mesh(),
        out_type=jax.ShapeDtypeStruct((e,), F32),
        scratch_types=[
            pltpu.VMEM((ECH,), jnp.int32),
            pltpu.VMEM((ECH,), jnp.int32),
            pltpu.VMEM((ECH, dtab), F32),
            pltpu.VMEM((ECH, dtab), F32),
            pltpu.VMEM((ECH,), F32),
        ])
    def sc_logits(z, gidx, out, ip_v, im_v, zp_v, zm_v, out_v):
        c = lax.axis_index("c")
        s = lax.axis_index("s")
        wbase = (c * NS + s) * per_w

        lane = lax.iota(jnp.int32, 16)
        perms = [jnp.bitwise_xor(lane, sh) for sh in (8, 4, 2, 1)]

        def body(k, carry):
            b = wbase + k * ECH
            pltpu.sync_copy(gidx.at[pl.ds(b, ECH)], ip_v)
            pltpu.sync_copy(gidx.at[pl.ds(e + b, ECH)], im_v)
            pltpu.sync_copy(z.at[ip_v], zp_v)
            pltpu.sync_copy(z.at[im_v], zm_v)
            for g in range(ECH // 16):
                res = jnp.zeros((16,), F32)
                for j in range(16):
                    r = g * 16 + j
                    acc = zp_v[r, pl.ds(0, 16)] * zm_v[r, pl.ds(0, 16)]
                    for q in range(1, nseg):
                        acc = acc + (zp_v[r, pl.ds(q * 16, 16)]
                                     * zm_v[r, pl.ds(q * 16, 16)])
                    for p in perms:
                        acc = acc + acc.at[p].get(
                            mode=lax.GatherScatterMode.PROMISE_IN_BOUNDS)
                    res = jnp.where(lane == j, acc, res)
                out_v[pl.ds(g * 16, 16)] = res
            pltpu.sync_copy(out_v, out.at[pl.ds(b, ECH)])
            return carry

        lax.fori_loop(0, nch, body, 0)

    return sc_logits


def kernel(x_member, x_provider, edge_index, W1_l, b1_l, W1_r,
           W2_l, b2_l, W2_r, Wd, bd):
    n_m, in_dim = x_member.shape
    n_p = x_provider.shape[0]
    n = n_m + n_p
    e = edge_index.shape[1]
    e2 = 2 * e
    hid = W1_l.shape[0]
    lat = W2_l.shape[0]
    n_half = n // 2

    x = jnp.concatenate([x_member, x_provider], axis=0)
    prov = edge_index[0]
    memb = edge_index[1]
    gidx = jnp.concatenate([prov + n_m, memb])
    sidx = jnp.concatenate([memb, prov])

    zrow = jnp.zeros((RCH, hid), F32)
    onerow = jnp.concatenate(
        [jnp.ones((ECH, 1), F32), jnp.zeros((ECH, hid - 1), F32)], axis=1)

    wcat1 = jnp.concatenate([W1_l.T, W1_r.T], axis=1)
    wcat2 = jnp.concatenate([W2_l.T, W2_r.T], axis=1)
    b1 = b1_l.reshape(1, hid)
    b2 = b2_l.reshape(1, lat)
    wdt = Wd.T
    bdr = bd.reshape(1, in_dim)

    cntf = _make_sc_cnt(n_half, e2)(sidx, zrow, onerow)
    pre1, xr1 = _tc_linear_pair(x, wcat1, hid)
    agg1 = _make_sc_agg(n_half, hid, e2)(pre1, gidx, sidx, zrow)
    pre2, hr2 = _tc_combine_linear(agg1, cntf, xr1, b1, wcat2, lat, True, hid)
    agg2 = _make_sc_agg(n_half, hid, e2)(pre2, gidx, sidx, zrow)
    z, x_hat = _tc_final(agg2, cntf, hr2, b2, wdt, bdr, hid)
    logits = _make_sc_edge_logits(e, hid, lat)(z, gidx)

    return (x_hat[:n_m], x_hat[n_m:], logits)


# speedup vs baseline: 6.7893x; 1.4289x over previous
import functools

import jax
import jax.numpy as jnp
from jax import lax
from jax.experimental import pallas as pl
from jax.experimental.pallas import tpu as pltpu
from jax.experimental.pallas import tpu_sc as plsc

F32 = jnp.float32

NC = 2
NS = 16
NW = NC * NS
ECH = 80
RCH = 128


def _mesh():
    return plsc.VectorSubcoreMesh(core_axis_name="c", subcore_axis_name="s",
                                  num_cores=NC, num_subcores=NS)


def _mm_kernel(x_ref, w_ref, a_out, b_out, split):
    o = jnp.dot(x_ref[:], w_ref[:], preferred_element_type=F32)
    a_out[:] = o[:, :split]
    b_out[:] = o[:, split:]


def _tc_linear_pair(x, wcat, split, blk=1000):
    r, k = x.shape
    m = wcat.shape[1]
    return pl.pallas_call(
        functools.partial(_mm_kernel, split=split),
        grid=(r // blk,),
        in_specs=[
            pl.BlockSpec((blk, k), lambda i: (i, 0)),
            pl.BlockSpec((k, m), lambda i: (0, 0)),
        ],
        out_specs=[
            pl.BlockSpec((blk, split), lambda i: (i, 0)),
            pl.BlockSpec((blk, m - split), lambda i: (i, 0)),
        ],
        out_shape=[
            jax.ShapeDtypeStruct((r, split), F32),
            jax.ShapeDtypeStruct((r, m - split), F32),
        ],
    )(x, wcat)


def _combine2_kernel(agg_ref, cnt_ref, xr_ref, b_ref, w_ref, a_out, b_out,
                     split, relu, pad_to):
    inv = 1.0 / jnp.maximum(cnt_ref[:, 0:1], 1.0)
    h = agg_ref[:] * inv + b_ref[:] + xr_ref[:]
    if relu:
        h = jnp.maximum(h, 0.0)
    o = jnp.dot(h, w_ref[:], preferred_element_type=F32)
    a = o[:, :split]
    if pad_to > split:
        a = jnp.concatenate(
            [a, jnp.zeros((a.shape[0], pad_to - split), F32)], axis=1)
    a_out[:] = a
    b_out[:] = o[:, split:]


def _tc_combine_linear(agg, cntf, xr, bvec, wcat, split, relu, pad_to,
                       blk=1000):
    r, d = agg.shape
    dh = xr.shape[1]
    m = wcat.shape[1]
    return pl.pallas_call(
        functools.partial(_combine2_kernel, split=split, relu=relu,
                          pad_to=pad_to),
        grid=(r // blk,),
        in_specs=[
            pl.BlockSpec((blk, d), lambda i: (i, 0)),
            pl.BlockSpec((blk, 128), lambda i: (i, 0)),
            pl.BlockSpec((blk, dh), lambda i: (i, 0)),
            pl.BlockSpec((1, dh), lambda i: (0, 0)),
            pl.BlockSpec((dh, m), lambda i: (0, 0)),
        ],
        out_specs=[
            pl.BlockSpec((blk, pad_to), lambda i: (i, 0)),
            pl.BlockSpec((blk, m - split), lambda i: (i, 0)),
        ],
        out_shape=[
            jax.ShapeDtypeStruct((r, pad_to), F32),
            jax.ShapeDtypeStruct((r, m - split), F32),
        ],
    )(agg, cntf, xr, bvec, wcat)


def _final_kernel(agg_ref, cnt_ref, hr_ref, b2_ref, wd_ref, bd_ref,
                  z_out, xh_out, d, pad_to):
    inv = 1.0 / jnp.maximum(cnt_ref[:, 0:1], 1.0)
    z = agg_ref[:, :d] * inv + b2_ref[:] + hr_ref[:]
    zp = jnp.concatenate([z, jnp.zeros((z.shape[0], pad_to - d), F32)], axis=1)
    z_out[:] = zp
    xh_out[:] = jnp.dot(z, wd_ref[:], preferred_element_type=F32) + bd_ref[:]


def _tc_final(agg2, cntf, hr2, b2, wdt, bd, pad_to, blk=1000):
    r, da = agg2.shape
    d = hr2.shape[1]
    m = wdt.shape[1]
    return pl.pallas_call(
        functools.partial(_final_kernel, d=d, pad_to=pad_to),
        grid=(r // blk,),
        in_specs=[
            pl.BlockSpec((blk, da), lambda i: (i, 0)),
            pl.BlockSpec((blk, 128), lambda i: (i, 0)),
            pl.BlockSpec((blk, d), lambda i: (i, 0)),
            pl.BlockSpec((1, d), lambda i: (0, 0)),
            pl.BlockSpec((d, m), lambda i: (0, 0)),
            pl.BlockSpec((1, m), lambda i: (0, 0)),
        ],
        out_specs=[
            pl.BlockSpec((blk, pad_to), lambda i: (i, 0)),
            pl.BlockSpec((blk, m), lambda i: (i, 0)),
        ],
        out_shape=[
            jax.ShapeDtypeStruct((r, pad_to), F32),
            jax.ShapeDtypeStruct((r, m), F32),
        ],
    )(agg2, cntf, hr2, b2, wdt, bd)


def _zero_acc(s, rows_v, acc, n_half):
    nz = (n_half + RCH * NS - 1) // (RCH * NS)
    maxoff = n_half - RCH

    def zbody(k, carry):
        off = jnp.minimum((k * NS + s) * RCH, maxoff)
        pltpu.sync_copy(rows_v, acc.at[pl.ds(off, RCH)])
        return carry

    lax.fori_loop(0, nz, zbody, 0)


def _flush_acc(c, s, stage_v, acc, out, n_half):
    nz = (n_half + RCH * NS - 1) // (RCH * NS)
    maxoff = n_half - RCH

    def fbody(k, carry):
        off = jnp.minimum((k * NS + s) * RCH, maxoff)
        pltpu.sync_copy(acc.at[pl.ds(off, RCH)], stage_v)
        pltpu.sync_copy(stage_v, out.at[pl.ds(c * n_half + off, RCH)])
        return carry

    lax.fori_loop(0, nz, fbody, 0)


def _make_sc_agg(n_half, d, e2):
    e_half = e2 // 2
    per_w = e_half // NS
    assert per_w % ECH == 0
    nch = per_w // ECH
    assert nch % 2 == 0

    @functools.partial(
        pl.kernel, mesh=_mesh(),
        out_type=jax.ShapeDtypeStruct((2 * n_half, d), F32),
        scratch_types=[
            pltpu.VMEM((2, ECH), jnp.int32),
            pltpu.VMEM((2, ECH), jnp.int32),
            pltpu.VMEM((ECH, d), F32),
            pltpu.VMEM((ECH, d), F32),
            pltpu.VMEM((RCH, d), F32),
            pltpu.VMEM_SHARED((n_half, d), F32),
            pltpu.SemaphoreType.DMA,
            pltpu.SemaphoreType.DMA,
        ])
    def sc_agg(table, pidx, zrow, agg_out,
               idx_v0, idx_v1, rows_v0, rows_v1, stage_v, acc, sem0, sem1):
        c = lax.axis_index("c")
        s = lax.axis_index("s")
        cbase = (c * NS + s) * nch

        pltpu.sync_copy(zrow, stage_v)
        _zero_acc(s, stage_v, acc, n_half)
        plsc.subcore_barrier()

        pltpu.sync_copy(pidx.at[cbase], idx_v0)
        pltpu.async_copy(table.at[idx_v0.at[0]], rows_v0, sem0)

        def body(j, carry):
            k1 = cbase + 2 * j + 1
            k2 = cbase + jnp.minimum(2 * j + 2, nch - 2)
            pltpu.sync_copy(pidx.at[k1], idx_v1)
            pltpu.async_copy(table.at[idx_v1.at[0]], rows_v1, sem1)
            pltpu.make_async_copy(table.at[idx_v0.at[0]], rows_v0, sem0).wait()
            pltpu.sync_copy(rows_v0, acc.at[idx_v0.at[1]], add=True)
            pltpu.sync_copy(pidx.at[k2], idx_v0)
            pltpu.async_copy(table.at[idx_v0.at[0]], rows_v0, sem0)
            pltpu.make_async_copy(table.at[idx_v1.at[0]], rows_v1, sem1).wait()
            pltpu.sync_copy(rows_v1, acc.at[idx_v1.at[1]], add=True)
            return carry

        lax.fori_loop(0, nch // 2, body, 0)
        pltpu.make_async_copy(table.at[idx_v0.at[0]], rows_v0, sem0).wait()
        plsc.subcore_barrier()
        _flush_acc(c, s, stage_v, acc, agg_out, n_half)

    return sc_agg


def _make_sc_cnt(n_half, e2):
    e_half = e2 // 2
    per_w = e_half // NS
    assert per_w % ECH == 0
    nch = per_w // ECH

    @functools.partial(
        pl.kernel, mesh=_mesh(),
        out_type=jax.ShapeDtypeStruct((2 * n_half, 128), F32),
        scratch_types=[
            pltpu.VMEM((ECH,), jnp.int32),
            pltpu.VMEM((ECH, 128), F32),
            pltpu.VMEM((RCH, 128), F32),
            pltpu.VMEM_SHARED((n_half, 128), F32),
        ])
    def sc_cnt(sidx, zrow, onerow, cnt_out, dst_v, ones_v, stage_v, acc):
        c = lax.axis_index("c")
        s = lax.axis_index("s")
        wbase = (c * NS + s) * per_w

        pltpu.sync_copy(zrow, stage_v)
        pltpu.sync_copy(onerow, ones_v)
        _zero_acc(s, stage_v, acc, n_half)
        plsc.subcore_barrier()

        def body(k, carry):
            b = wbase + k * ECH
            pltpu.sync_copy(sidx.at[pl.ds(b, ECH)], dst_v)
            pltpu.sync_copy(ones_v, acc.at[dst_v], add=True)
            return carry

        lax.fori_loop(0, nch, body, 0)
        plsc.subcore_barrier()
        _flush_acc(c, s, stage_v, acc, cnt_out, n_half)

    return sc_cnt


def _make_sc_edge_logits(e, dtab, d):
    per_w = e // NW
    assert per_w % ECH == 0
    nch = per_w // ECH
    assert d % 16 == 0
    nseg = d // 16

    @functools.partial(
        pl.kernel, mesh=_mesh(),
        out_type=jax.ShapeDtypeStruct((e,), F32),
        scratch_types=[
            pltpu.VMEM((ECH,), jnp.int32),
            pltpu.VMEM((ECH,), jnp.int32),
            pltpu.VMEM((ECH, dtab), F32),
            pltpu.VMEM((ECH, dtab), F32),
            pltpu.VMEM((ECH,), F32),
        ])
    def sc_logits(z, gidx, out, ip_v, im_v, zp_v, zm_v, out_v):
        c = lax.axis_index("c")
        s = lax.axis_index("s")
        wbase = (c * NS + s) * per_w

        lane = lax.iota(jnp.int32, 16)
        perms = [jnp.bitwise_xor(lane, sh) for sh in (8, 4, 2, 1)]

        def body(k, carry):
            b = wbase + k * ECH
            pltpu.sync_copy(gidx.at[pl.ds(b, ECH)], ip_v)
            pltpu.sync_copy(gidx.at[pl.ds(e + b, ECH)], im_v)
            pltpu.sync_copy(z.at[ip_v], zp_v)
            pltpu.sync_copy(z.at[im_v], zm_v)
            for g in range(ECH // 16):
                res = jnp.zeros((16,), F32)
                for j in range(16):
                    r = g * 16 + j
                    acc = zp_v[r, pl.ds(0, 16)] * zm_v[r, pl.ds(0, 16)]
                    for q in range(1, nseg):
                        acc = acc + (zp_v[r, pl.ds(q * 16, 16)]
                                     * zm_v[r, pl.ds(q * 16, 16)])
                    for p in perms:
                        acc = acc + acc.at[p].get(
                            mode=lax.GatherScatterMode.PROMISE_IN_BOUNDS)
                    res = jnp.where(lane == j, acc, res)
                out_v[pl.ds(g * 16, 16)] = res
            pltpu.sync_copy(out_v, out.at[pl.ds(b, ECH)])
            return carry

        lax.fori_loop(0, nch, body, 0)

    return sc_logits


def kernel(x_member, x_provider, edge_index, W1_l, b1_l, W1_r,
           W2_l, b2_l, W2_r, Wd, bd):
    n_m, in_dim = x_member.shape
    n_p = x_provider.shape[0]
    n = n_m + n_p
    e = edge_index.shape[1]
    e2 = 2 * e
    hid = W1_l.shape[0]
    lat = W2_l.shape[0]
    n_half = n // 2

    x = jnp.concatenate([x_member, x_provider], axis=0)
    prov = edge_index[0]
    memb = edge_index[1]
    gidx = jnp.concatenate([prov + n_m, memb])
    sidx = jnp.concatenate([memb, prov])

    zrow = jnp.zeros((RCH, hid), F32)
    onerow = jnp.concatenate(
        [jnp.ones((ECH, 1), F32), jnp.zeros((ECH, hid - 1), F32)], axis=1)

    wcat1 = jnp.concatenate([W1_l.T, W1_r.T], axis=1)
    wcat2 = jnp.concatenate([W2_l.T, W2_r.T], axis=1)
    b1 = b1_l.reshape(1, hid)
    b2 = b2_l.reshape(1, lat)
    wdt = Wd.T
    bdr = bd.reshape(1, in_dim)

    cntf = _make_sc_cnt(n_half, e2)(sidx, zrow, onerow)
    nch_tot = e2 // ECH
    pidx = jnp.stack([gidx.reshape(nch_tot, ECH),
                      sidx.reshape(nch_tot, ECH)], axis=1)
    pre1, xr1 = _tc_linear_pair(x, wcat1, hid)
    agg1 = _make_sc_agg(n_half, hid, e2)(pre1, pidx, zrow)
    pre2, hr2 = _tc_combine_linear(agg1, cntf, xr1, b1, wcat2, lat, True, hid)
    agg2 = _make_sc_agg(n_half, hid, e2)(pre2, pidx, zrow)
    z, x_hat = _tc_final(agg2, cntf, hr2, b2, wdt, bdr, hid)
    logits = _make_sc_edge_logits(e, hid, lat)(z, gidx)

    return (x_hat[:n_m], x_hat[n_m:], logits)


# speedup vs baseline: 7.6961x; 1.1336x over previous
import functools

import jax
import jax.numpy as jnp
from jax import lax
from jax.experimental import pallas as pl
from jax.experimental.pallas import tpu as pltpu
from jax.experimental.pallas import tpu_sc as plsc

F32 = jnp.float32

NC = 2
NS = 16
NW = NC * NS
ECH = 80
RCH = 128


def _mesh():
    return plsc.VectorSubcoreMesh(core_axis_name="c", subcore_axis_name="s",
                                  num_cores=NC, num_subcores=NS)


def _mm_kernel(x_ref, w_ref, a_out, b_out, split):
    o = jnp.dot(x_ref[:], w_ref[:], preferred_element_type=F32)
    a_out[:] = o[:, :split]
    b_out[:] = o[:, split:]


def _tc_linear_pair(x, wcat, split, blk=1000):
    r, k = x.shape
    m = wcat.shape[1]
    return pl.pallas_call(
        functools.partial(_mm_kernel, split=split),
        grid=(r // blk,),
        in_specs=[
            pl.BlockSpec((blk, k), lambda i: (i, 0)),
            pl.BlockSpec((k, m), lambda i: (0, 0)),
        ],
        out_specs=[
            pl.BlockSpec((blk, split), lambda i: (i, 0)),
            pl.BlockSpec((blk, m - split), lambda i: (i, 0)),
        ],
        out_shape=[
            jax.ShapeDtypeStruct((r, split), F32),
            jax.ShapeDtypeStruct((r, m - split), F32),
        ],
    )(x, wcat)


def _combine2_kernel(agg_ref, cnt_ref, xr_ref, b_ref, w_ref, a_out, b_out,
                     split, relu, pad_to):
    inv = 1.0 / jnp.maximum(cnt_ref[:, 0:1], 1.0)
    h = agg_ref[:] * inv + b_ref[:] + xr_ref[:]
    if relu:
        h = jnp.maximum(h, 0.0)
    o = jnp.dot(h, w_ref[:], preferred_element_type=F32)
    a = o[:, :split]
    if pad_to > split:
        a = jnp.concatenate(
            [a, jnp.zeros((a.shape[0], pad_to - split), F32)], axis=1)
    a_out[:] = a
    b_out[:] = o[:, split:]


def _tc_combine_linear(agg, cntf, xr, bvec, wcat, split, relu, pad_to,
                       blk=1000):
    r, d = agg.shape
    dh = xr.shape[1]
    m = wcat.shape[1]
    return pl.pallas_call(
        functools.partial(_combine2_kernel, split=split, relu=relu,
                          pad_to=pad_to),
        grid=(r // blk,),
        in_specs=[
            pl.BlockSpec((blk, d), lambda i: (i, 0)),
            pl.BlockSpec((blk, 128), lambda i: (i, 0)),
            pl.BlockSpec((blk, dh), lambda i: (i, 0)),
            pl.BlockSpec((1, dh), lambda i: (0, 0)),
            pl.BlockSpec((dh, m), lambda i: (0, 0)),
        ],
        out_specs=[
            pl.BlockSpec((blk, pad_to), lambda i: (i, 0)),
            pl.BlockSpec((blk, m - split), lambda i: (i, 0)),
        ],
        out_shape=[
            jax.ShapeDtypeStruct((r, pad_to), F32),
            jax.ShapeDtypeStruct((r, m - split), F32),
        ],
    )(agg, cntf, xr, bvec, wcat)


def _final_kernel(agg_ref, cnt_ref, hr_ref, b2_ref, wd_ref, bd_ref,
                  z_out, xh_out, d, pad_to):
    inv = 1.0 / jnp.maximum(cnt_ref[:, 0:1], 1.0)
    z = agg_ref[:, :d] * inv + b2_ref[:] + hr_ref[:]
    zp = jnp.concatenate([z, jnp.zeros((z.shape[0], pad_to - d), F32)], axis=1)
    z_out[:] = zp
    xh_out[:] = jnp.dot(z, wd_ref[:], preferred_element_type=F32) + bd_ref[:]


def _tc_final(agg2, cntf, hr2, b2, wdt, bd, pad_to, blk=1000):
    r, da = agg2.shape
    d = hr2.shape[1]
    m = wdt.shape[1]
    return pl.pallas_call(
        functools.partial(_final_kernel, d=d, pad_to=pad_to),
        grid=(r // blk,),
        in_specs=[
            pl.BlockSpec((blk, da), lambda i: (i, 0)),
            pl.BlockSpec((blk, 128), lambda i: (i, 0)),
            pl.BlockSpec((blk, d), lambda i: (i, 0)),
            pl.BlockSpec((1, d), lambda i: (0, 0)),
            pl.BlockSpec((d, m), lambda i: (0, 0)),
            pl.BlockSpec((1, m), lambda i: (0, 0)),
        ],
        out_specs=[
            pl.BlockSpec((blk, pad_to), lambda i: (i, 0)),
            pl.BlockSpec((blk, m), lambda i: (i, 0)),
        ],
        out_shape=[
            jax.ShapeDtypeStruct((r, pad_to), F32),
            jax.ShapeDtypeStruct((r, m), F32),
        ],
    )(agg2, cntf, hr2, b2, wdt, bd)


def _zero_acc(s, rows_v, acc, n_half):
    nz = (n_half + RCH * NS - 1) // (RCH * NS)
    maxoff = n_half - RCH

    def zbody(k, carry):
        off = jnp.minimum((k * NS + s) * RCH, maxoff)
        pltpu.sync_copy(rows_v, acc.at[pl.ds(off, RCH)])
        return carry

    lax.fori_loop(0, nz, zbody, 0)


def _flush_acc(c, s, stage_v, acc, out, n_half):
    nz = (n_half + RCH * NS - 1) // (RCH * NS)
    maxoff = n_half - RCH

    def fbody(k, carry):
        off = jnp.minimum((k * NS + s) * RCH, maxoff)
        pltpu.sync_copy(acc.at[pl.ds(off, RCH)], stage_v)
        pltpu.sync_copy(stage_v, out.at[pl.ds(c * n_half + off, RCH)])
        return carry

    lax.fori_loop(0, nz, fbody, 0)


def _make_sc_agg(n_half, d, e2):
    e_half = e2 // 2
    per_w = e_half // NS
    assert per_w % ECH == 0
    nch = per_w // ECH
    assert nch % 2 == 0

    @functools.partial(
        pl.kernel, mesh=_mesh(),
        out_type=jax.ShapeDtypeStruct((2 * n_half, d), F32),
        scratch_types=[
            pltpu.VMEM((2, ECH), jnp.int32),
            pltpu.VMEM((2, ECH), jnp.int32),
            pltpu.VMEM((ECH, d), F32),
            pltpu.VMEM((ECH, d), F32),
            pltpu.VMEM((RCH, d), F32),
            pltpu.VMEM_SHARED((n_half, d), F32),
            pltpu.SemaphoreType.DMA,
            pltpu.SemaphoreType.DMA,
        ])
    def sc_agg(table, pidx, zrow, agg_out,
               idx_v0, idx_v1, rows_v0, rows_v1, stage_v, acc, sem0, sem1):
        c = lax.axis_index("c")
        s = lax.axis_index("s")
        cbase = (c * NS + s) * nch

        pltpu.sync_copy(zrow, stage_v)
        _zero_acc(s, stage_v, acc, n_half)
        plsc.subcore_barrier()

        pltpu.sync_copy(pidx.at[cbase], idx_v0)
        pltpu.async_copy(table.at[idx_v0.at[0]], rows_v0, sem0)

        def body(j, carry):
            k1 = cbase + 2 * j + 1
            k2 = cbase + jnp.minimum(2 * j + 2, nch - 2)
            pltpu.sync_copy(pidx.at[k1], idx_v1)
            pltpu.async_copy(table.at[idx_v1.at[0]], rows_v1, sem1)
            pltpu.make_async_copy(table.at[idx_v0.at[0]], rows_v0, sem0).wait()
            pltpu.sync_copy(rows_v0, acc.at[idx_v0.at[1]], add=True)
            pltpu.sync_copy(pidx.at[k2], idx_v0)
            pltpu.async_copy(table.at[idx_v0.at[0]], rows_v0, sem0)
            pltpu.make_async_copy(table.at[idx_v1.at[0]], rows_v1, sem1).wait()
            pltpu.sync_copy(rows_v1, acc.at[idx_v1.at[1]], add=True)
            return carry

        lax.fori_loop(0, nch // 2, body, 0)
        pltpu.make_async_copy(table.at[idx_v0.at[0]], rows_v0, sem0).wait()
        plsc.subcore_barrier()
        _flush_acc(c, s, stage_v, acc, agg_out, n_half)

    return sc_agg


def _make_sc_cnt(n_half, e2):
    e_half = e2 // 2
    per_w = e_half // NS
    assert per_w % ECH == 0
    nch = per_w // ECH

    @functools.partial(
        pl.kernel, mesh=_mesh(),
        out_type=jax.ShapeDtypeStruct((2 * n_half, 128), F32),
        scratch_types=[
            pltpu.VMEM((ECH,), jnp.int32),
            pltpu.VMEM((ECH, 128), F32),
            pltpu.VMEM((RCH, 128), F32),
            pltpu.VMEM_SHARED((n_half, 128), F32),
        ])
    def sc_cnt(sidx, zrow, onerow, cnt_out, dst_v, ones_v, stage_v, acc):
        c = lax.axis_index("c")
        s = lax.axis_index("s")
        wbase = (c * NS + s) * per_w

        pltpu.sync_copy(zrow, stage_v)
        pltpu.sync_copy(onerow, ones_v)
        _zero_acc(s, stage_v, acc, n_half)
        plsc.subcore_barrier()

        def body(k, carry):
            b = wbase + k * ECH
            pltpu.sync_copy(sidx.at[pl.ds(b, ECH)], dst_v)
            pltpu.sync_copy(ones_v, acc.at[dst_v], add=True)
            return carry

        lax.fori_loop(0, nch, body, 0)
        plsc.subcore_barrier()
        _flush_acc(c, s, stage_v, acc, cnt_out, n_half)

    return sc_cnt


def _make_sc_edge_logits(e, dtab, d):
    per_w = e // NW
    assert per_w % ECH == 0
    nch = per_w // ECH
    assert nch % 2 == 1
    assert d % 16 == 0
    nseg = d // 16

    @functools.partial(
        pl.kernel, mesh=_mesh(),
        out_type=jax.ShapeDtypeStruct((e,), F32),
        scratch_types=[
            pltpu.VMEM((2, ECH), jnp.int32),
            pltpu.VMEM((2, ECH), jnp.int32),
            pltpu.VMEM((ECH, dtab), F32),
            pltpu.VMEM((ECH, dtab), F32),
            pltpu.VMEM((ECH, dtab), F32),
            pltpu.VMEM((ECH, dtab), F32),
            pltpu.VMEM((ECH,), F32),
            pltpu.SemaphoreType.DMA,
            pltpu.SemaphoreType.DMA,
            pltpu.SemaphoreType.DMA,
            pltpu.SemaphoreType.DMA,
        ])
    def sc_logits(z, lidx, out, idx_v0, idx_v1, zp_v0, zm_v0, zp_v1, zm_v1,
                  out_v, semp0, semm0, semp1, semm1):
        c = lax.axis_index("c")
        s = lax.axis_index("s")
        cbase = (c * NS + s) * nch

        lane = lax.iota(jnp.int32, 16)
        perms = [jnp.bitwise_xor(lane, sh) for sh in (8, 4, 2, 1)]

        def dots(zp_v, zm_v):
            for g in range(ECH // 16):
                res = jnp.zeros((16,), F32)
                for j in range(16):
                    r = g * 16 + j
                    acc = zp_v[r, pl.ds(0, 16)] * zm_v[r, pl.ds(0, 16)]
                    for q in range(1, nseg):
                        acc = acc + (zp_v[r, pl.ds(q * 16, 16)]
                                     * zm_v[r, pl.ds(q * 16, 16)])
                    for p in perms:
                        acc = acc + acc.at[p].get(
                            mode=lax.GatherScatterMode.PROMISE_IN_BOUNDS)
                    res = jnp.where(lane == j, acc, res)
                out_v[pl.ds(g * 16, 16)] = res

        def fire(idx_v, zp_v, zm_v, sp, sm):
            pltpu.async_copy(z.at[idx_v.at[0]], zp_v, sp)
            pltpu.async_copy(z.at[idx_v.at[1]], zm_v, sm)

        def drain_compute(k, idx_v, zp_v, zm_v, sp, sm):
            pltpu.make_async_copy(z.at[idx_v.at[0]], zp_v, sp).wait()
            pltpu.make_async_copy(z.at[idx_v.at[1]], zm_v, sm).wait()
            dots(zp_v, zm_v)
            pltpu.sync_copy(out_v, out.at[pl.ds((cbase + k) * ECH, ECH)])

        pltpu.sync_copy(lidx.at[cbase], idx_v0)
        fire(idx_v0, zp_v0, zm_v0, semp0, semm0)

        def body(j, carry):
            pltpu.sync_copy(lidx.at[cbase + 2 * j + 1], idx_v1)
            fire(idx_v1, zp_v1, zm_v1, semp1, semm1)
            drain_compute(2 * j, idx_v0, zp_v0, zm_v0, semp0, semm0)
            pltpu.sync_copy(lidx.at[cbase + 2 * j + 2], idx_v0)
            fire(idx_v0, zp_v0, zm_v0, semp0, semm0)
            drain_compute(2 * j + 1, idx_v1, zp_v1, zm_v1, semp1, semm1)
            return carry

        lax.fori_loop(0, nch // 2, body, 0)
        drain_compute(nch - 1, idx_v0, zp_v0, zm_v0, semp0, semm0)

    return sc_logits


def kernel(x_member, x_provider, edge_index, W1_l, b1_l, W1_r,
           W2_l, b2_l, W2_r, Wd, bd):
    n_m, in_dim = x_member.shape
    n_p = x_provider.shape[0]
    n = n_m + n_p
    e = edge_index.shape[1]
    e2 = 2 * e
    hid = W1_l.shape[0]
    lat = W2_l.shape[0]
    n_half = n // 2

    x = jnp.concatenate([x_member, x_provider], axis=0)
    prov = edge_index[0]
    memb = edge_index[1]
    gidx = jnp.concatenate([prov + n_m, memb])
    sidx = jnp.concatenate([memb, prov])

    zrow = jnp.zeros((RCH, hid), F32)
    onerow = jnp.concatenate(
        [jnp.ones((ECH, 1), F32), jnp.zeros((ECH, hid - 1), F32)], axis=1)

    wcat1 = jnp.concatenate([W1_l.T, W1_r.T], axis=1)
    wcat2 = jnp.concatenate([W2_l.T, W2_r.T], axis=1)
    b1 = b1_l.reshape(1, hid)
    b2 = b2_l.reshape(1, lat)
    wdt = Wd.T
    bdr = bd.reshape(1, in_dim)

    cntf = _make_sc_cnt(n_half, e2)(sidx, zrow, onerow)
    nch_tot = e2 // ECH
    pidx = jnp.stack([gidx.reshape(nch_tot, ECH),
                      sidx.reshape(nch_tot, ECH)], axis=1)
    pre1, xr1 = _tc_linear_pair(x, wcat1, hid)
    agg1 = _make_sc_agg(n_half, hid, e2)(pre1, pidx, zrow)
    pre2, hr2 = _tc_combine_linear(agg1, cntf, xr1, b1, wcat2, lat, True, hid)
    agg2 = _make_sc_agg(n_half, hid, e2)(pre2, pidx, zrow)
    z, x_hat = _tc_final(agg2, cntf, hr2, b2, wdt, bdr, hid)
    lidx = jnp.stack([(prov + n_m).reshape(e // ECH, ECH),
                      memb.reshape(e // ECH, ECH)], axis=1)
    logits = _make_sc_edge_logits(e, hid, lat)(z, lidx)

    return (x_hat[:n_m], x_hat[n_m:], logits)


# speedup vs baseline: 8.3053x; 1.0792x over previous
import functools

import jax
import jax.numpy as jnp
from jax import lax
from jax.experimental import pallas as pl
from jax.experimental.pallas import tpu as pltpu
from jax.experimental.pallas import tpu_sc as plsc

F32 = jnp.float32

NC = 2
NS = 16
NW = NC * NS
ECH = 80
RCH = 128


def _mesh():
    return plsc.VectorSubcoreMesh(core_axis_name="c", subcore_axis_name="s",
                                  num_cores=NC, num_subcores=NS)


def _mm_kernel(x_ref, w_ref, a_out, b_out, split):
    o = jnp.dot(x_ref[:], w_ref[:], preferred_element_type=F32)
    a_out[:] = o[:, :split]
    b_out[:] = o[:, split:]


def _tc_linear_pair(x, wcat, split, blk=1000):
    r, k = x.shape
    m = wcat.shape[1]
    return pl.pallas_call(
        functools.partial(_mm_kernel, split=split),
        grid=(r // blk,),
        in_specs=[
            pl.BlockSpec((blk, k), lambda i: (i, 0)),
            pl.BlockSpec((k, m), lambda i: (0, 0)),
        ],
        out_specs=[
            pl.BlockSpec((blk, split), lambda i: (i, 0)),
            pl.BlockSpec((blk, m - split), lambda i: (i, 0)),
        ],
        out_shape=[
            jax.ShapeDtypeStruct((r, split), F32),
            jax.ShapeDtypeStruct((r, m - split), F32),
        ],
    )(x, wcat)


def _combine2_kernel(agg_ref, cnt_ref, xr_ref, b_ref, w_ref, a_out, b_out,
                     split, relu, pad_to):
    inv = 1.0 / jnp.maximum(cnt_ref[:, 0:1], 1.0)
    h = agg_ref[:] * inv + b_ref[:] + xr_ref[:]
    if relu:
        h = jnp.maximum(h, 0.0)
    o = jnp.dot(h, w_ref[:], preferred_element_type=F32)
    a = o[:, :split]
    if pad_to > split:
        a = jnp.concatenate(
            [a, jnp.zeros((a.shape[0], pad_to - split), F32)], axis=1)
    a_out[:] = a
    b_out[:] = o[:, split:]


def _tc_combine_linear(agg, cntf, xr, bvec, wcat, split, relu, pad_to,
                       blk=1000):
    r, d = agg.shape
    dh = xr.shape[1]
    m = wcat.shape[1]
    return pl.pallas_call(
        functools.partial(_combine2_kernel, split=split, relu=relu,
                          pad_to=pad_to),
        grid=(r // blk,),
        in_specs=[
            pl.BlockSpec((blk, d), lambda i: (i, 0)),
            pl.BlockSpec((blk, 128), lambda i: (i, 0)),
            pl.BlockSpec((blk, dh), lambda i: (i, 0)),
            pl.BlockSpec((1, dh), lambda i: (0, 0)),
            pl.BlockSpec((dh, m), lambda i: (0, 0)),
        ],
        out_specs=[
            pl.BlockSpec((blk, pad_to), lambda i: (i, 0)),
            pl.BlockSpec((blk, m - split), lambda i: (i, 0)),
        ],
        out_shape=[
            jax.ShapeDtypeStruct((r, pad_to), F32),
            jax.ShapeDtypeStruct((r, m - split), F32),
        ],
    )(agg, cntf, xr, bvec, wcat)


def _final_kernel(agg_ref, cnt_ref, hr_ref, b2_ref, wd_ref, bd_ref,
                  z_out, xh_out, d, pad_to):
    inv = 1.0 / jnp.maximum(cnt_ref[:, 0:1], 1.0)
    z = agg_ref[:, :d] * inv + b2_ref[:] + hr_ref[:]
    zp = jnp.concatenate([z, jnp.zeros((z.shape[0], pad_to - d), F32)], axis=1)
    z_out[:] = zp
    xh_out[:] = jnp.dot(z, wd_ref[:], preferred_element_type=F32) + bd_ref[:]


def _tc_final(agg2, cntf, hr2, b2, wdt, bd, pad_to, blk=1000):
    r, da = agg2.shape
    d = hr2.shape[1]
    m = wdt.shape[1]
    return pl.pallas_call(
        functools.partial(_final_kernel, d=d, pad_to=pad_to),
        grid=(r // blk,),
        in_specs=[
            pl.BlockSpec((blk, da), lambda i: (i, 0)),
            pl.BlockSpec((blk, 128), lambda i: (i, 0)),
            pl.BlockSpec((blk, d), lambda i: (i, 0)),
            pl.BlockSpec((1, d), lambda i: (0, 0)),
            pl.BlockSpec((d, m), lambda i: (0, 0)),
            pl.BlockSpec((1, m), lambda i: (0, 0)),
        ],
        out_specs=[
            pl.BlockSpec((blk, pad_to), lambda i: (i, 0)),
            pl.BlockSpec((blk, m), lambda i: (i, 0)),
        ],
        out_shape=[
            jax.ShapeDtypeStruct((r, pad_to), F32),
            jax.ShapeDtypeStruct((r, m), F32),
        ],
    )(agg2, cntf, hr2, b2, wdt, bd)


def _zero_acc(s, rows_v, acc, n_half):
    nz = (n_half + RCH * NS - 1) // (RCH * NS)
    maxoff = n_half - RCH

    def zbody(k, carry):
        off = jnp.minimum((k * NS + s) * RCH, maxoff)
        pltpu.sync_copy(rows_v, acc.at[pl.ds(off, RCH)])
        return carry

    lax.fori_loop(0, nz, zbody, 0)


def _flush_acc(c, s, stage_v, acc, out, n_half):
    nz = (n_half + RCH * NS - 1) // (RCH * NS)
    maxoff = n_half - RCH

    def fbody(k, carry):
        off = jnp.minimum((k * NS + s) * RCH, maxoff)
        pltpu.sync_copy(acc.at[pl.ds(off, RCH)], stage_v)
        pltpu.sync_copy(stage_v, out.at[pl.ds(c * n_half + off, RCH)])
        return carry

    lax.fori_loop(0, nz, fbody, 0)


def _make_sc_agg(n_half, d, e2):
    e_half = e2 // 2
    per_w = e_half // NS
    assert per_w % ECH == 0
    nch = per_w // ECH
    assert nch % 2 == 0

    @functools.partial(
        pl.kernel, mesh=_mesh(),
        out_type=jax.ShapeDtypeStruct((2 * n_half, d), F32),
        scratch_types=[
            pltpu.VMEM((2, ECH), jnp.int32),
            pltpu.VMEM((2, ECH), jnp.int32),
            pltpu.VMEM((ECH, d), F32),
            pltpu.VMEM((ECH, d), F32),
            pltpu.VMEM((RCH, d), F32),
            pltpu.VMEM_SHARED((n_half, d), F32),
            pltpu.SemaphoreType.DMA,
            pltpu.SemaphoreType.DMA,
        ])
    def sc_agg(table, pidx, zrow, agg_out,
               idx_v0, idx_v1, rows_v0, rows_v1, stage_v, acc, sem0, sem1):
        c = lax.axis_index("c")
        s = lax.axis_index("s")
        cbase = (c * NS + s) * nch

        pltpu.sync_copy(zrow, stage_v)
        _zero_acc(s, stage_v, acc, n_half)
        plsc.subcore_barrier()

        pltpu.sync_copy(pidx.at[cbase], idx_v0)
        pltpu.async_copy(table.at[idx_v0.at[0]], rows_v0, sem0)

        def body(j, carry):
            k1 = cbase + 2 * j + 1
            k2 = cbase + jnp.minimum(2 * j + 2, nch - 2)
            pltpu.sync_copy(pidx.at[k1], idx_v1)
            pltpu.async_copy(table.at[idx_v1.at[0]], rows_v1, sem1)
            pltpu.make_async_copy(table.at[idx_v0.at[0]], rows_v0, sem0).wait()
            pltpu.sync_copy(rows_v0, acc.at[idx_v0.at[1]], add=True)
            pltpu.sync_copy(pidx.at[k2], idx_v0)
            pltpu.async_copy(table.at[idx_v0.at[0]], rows_v0, sem0)
            pltpu.make_async_copy(table.at[idx_v1.at[0]], rows_v1, sem1).wait()
            pltpu.sync_copy(rows_v1, acc.at[idx_v1.at[1]], add=True)
            return carry

        lax.fori_loop(0, nch // 2, body, 0)
        pltpu.make_async_copy(table.at[idx_v0.at[0]], rows_v0, sem0).wait()
        plsc.subcore_barrier()
        _flush_acc(c, s, stage_v, acc, agg_out, n_half)

    return sc_agg


def _make_sc_cnt(n_half, e2):
    e_half = e2 // 2
    per_w = e_half // NS
    assert per_w % ECH == 0
    nch = per_w // ECH

    assert nch % 2 == 0

    @functools.partial(
        pl.kernel, mesh=_mesh(),
        out_type=jax.ShapeDtypeStruct((2 * n_half, 128), F32),
        scratch_types=[
            pltpu.VMEM((2, ECH), jnp.int32),
            pltpu.VMEM((2, ECH), jnp.int32),
            pltpu.VMEM((ECH, 128), F32),
            pltpu.VMEM((RCH, 128), F32),
            pltpu.VMEM_SHARED((n_half, 128), F32),
            pltpu.SemaphoreType.DMA,
            pltpu.SemaphoreType.DMA,
        ])
    def sc_cnt(pidx, zrow, onerow, cnt_out, idx_v0, idx_v1, ones_v, stage_v,
               acc, semi0, semi1):
        c = lax.axis_index("c")
        s = lax.axis_index("s")
        cbase = (c * NS + s) * nch

        pltpu.sync_copy(zrow, stage_v)
        pltpu.sync_copy(onerow, ones_v)
        _zero_acc(s, stage_v, acc, n_half)
        plsc.subcore_barrier()

        pltpu.async_copy(pidx.at[cbase], idx_v0, semi0)

        def body(j, carry):
            k2 = cbase + jnp.minimum(2 * j + 2, nch - 2)
            pltpu.make_async_copy(pidx.at[cbase], idx_v0, semi0).wait()
            pltpu.async_copy(pidx.at[cbase + 2 * j + 1], idx_v1, semi1)
            pltpu.sync_copy(ones_v, acc.at[idx_v0.at[1]], add=True)
            pltpu.make_async_copy(pidx.at[cbase], idx_v1, semi1).wait()
            pltpu.async_copy(pidx.at[k2], idx_v0, semi0)
            pltpu.sync_copy(ones_v, acc.at[idx_v1.at[1]], add=True)
            return carry

        lax.fori_loop(0, nch // 2, body, 0)
        pltpu.make_async_copy(pidx.at[cbase], idx_v0, semi0).wait()
        plsc.subcore_barrier()
        _flush_acc(c, s, stage_v, acc, cnt_out, n_half)

    return sc_cnt


def _make_sc_edge_logits(e, dtab, d):
    per_w = e // NW
    assert per_w % ECH == 0
    nch = per_w // ECH
    assert nch % 2 == 1
    assert d % 16 == 0
    nseg = d // 16

    @functools.partial(
        pl.kernel, mesh=_mesh(),
        out_type=jax.ShapeDtypeStruct((e,), F32),
        scratch_types=[
            pltpu.VMEM((2, ECH), jnp.int32),
            pltpu.VMEM((2, ECH), jnp.int32),
            pltpu.VMEM((ECH, dtab), F32),
            pltpu.VMEM((ECH, dtab), F32),
            pltpu.VMEM((ECH, dtab), F32),
            pltpu.VMEM((ECH, dtab), F32),
            pltpu.VMEM((ECH,), F32),
            pltpu.SemaphoreType.DMA,
            pltpu.SemaphoreType.DMA,
            pltpu.SemaphoreType.DMA,
            pltpu.SemaphoreType.DMA,
        ])
    def sc_logits(z, lidx, out, idx_v0, idx_v1, zp_v0, zm_v0, zp_v1, zm_v1,
                  out_v, semp0, semm0, semp1, semm1):
        c = lax.axis_index("c")
        s = lax.axis_index("s")
        cbase = (c * NS + s) * nch

        lane = lax.iota(jnp.int32, 16)
        perms = [jnp.bitwise_xor(lane, sh) for sh in (8, 4, 2, 1)]

        def dots(zp_v, zm_v):
            for g in range(ECH // 16):
                res = jnp.zeros((16,), F32)
                for j in range(16):
                    r = g * 16 + j
                    acc = zp_v[r, pl.ds(0, 16)] * zm_v[r, pl.ds(0, 16)]
                    for q in range(1, nseg):
                        acc = acc + (zp_v[r, pl.ds(q * 16, 16)]
                                     * zm_v[r, pl.ds(q * 16, 16)])
                    for p in perms:
                        acc = acc + acc.at[p].get(
                            mode=lax.GatherScatterMode.PROMISE_IN_BOUNDS)
                    res = jnp.where(lane == j, acc, res)
                out_v[pl.ds(g * 16, 16)] = res

        def fire(idx_v, zp_v, zm_v, sp, sm):
            pltpu.async_copy(z.at[idx_v.at[0]], zp_v, sp)
            pltpu.async_copy(z.at[idx_v.at[1]], zm_v, sm)

        def drain_compute(k, idx_v, zp_v, zm_v, sp, sm):
            pltpu.make_async_copy(z.at[idx_v.at[0]], zp_v, sp).wait()
            pltpu.make_async_copy(z.at[idx_v.at[1]], zm_v, sm).wait()
            dots(zp_v, zm_v)
            pltpu.sync_copy(out_v, out.at[pl.ds((cbase + k) * ECH, ECH)])

        pltpu.sync_copy(lidx.at[cbase], idx_v0)
        fire(idx_v0, zp_v0, zm_v0, semp0, semm0)

        def body(j, carry):
            pltpu.sync_copy(lidx.at[cbase + 2 * j + 1], idx_v1)
            fire(idx_v1, zp_v1, zm_v1, semp1, semm1)
            drain_compute(2 * j, idx_v0, zp_v0, zm_v0, semp0, semm0)
            pltpu.sync_copy(lidx.at[cbase + 2 * j + 2], idx_v0)
            fire(idx_v0, zp_v0, zm_v0, semp0, semm0)
            drain_compute(2 * j + 1, idx_v1, zp_v1, zm_v1, semp1, semm1)
            return carry

        lax.fori_loop(0, nch // 2, body, 0)
        drain_compute(nch - 1, idx_v0, zp_v0, zm_v0, semp0, semm0)

    return sc_logits


def kernel(x_member, x_provider, edge_index, W1_l, b1_l, W1_r,
           W2_l, b2_l, W2_r, Wd, bd):
    n_m, in_dim = x_member.shape
    n_p = x_provider.shape[0]
    n = n_m + n_p
    e = edge_index.shape[1]
    e2 = 2 * e
    hid = W1_l.shape[0]
    lat = W2_l.shape[0]
    n_half = n // 2

    x = jnp.concatenate([x_member, x_provider], axis=0)
    prov = edge_index[0]
    memb = edge_index[1]
    gidx = jnp.concatenate([prov + n_m, memb])
    sidx = jnp.concatenate([memb, prov])

    zrow = jnp.zeros((RCH, hid), F32)
    onerow = jnp.concatenate(
        [jnp.ones((ECH, 1), F32), jnp.zeros((ECH, hid - 1), F32)], axis=1)

    wcat1 = jnp.concatenate([W1_l.T, W1_r.T], axis=1)
    wcat2 = jnp.concatenate([W2_l.T, W2_r.T], axis=1)
    b1 = b1_l.reshape(1, hid)
    b2 = b2_l.reshape(1, lat)
    wdt = Wd.T
    bdr = bd.reshape(1, in_dim)

    nch_tot = e2 // ECH
    pidx = jnp.stack([gidx.reshape(nch_tot, ECH),
                      sidx.reshape(nch_tot, ECH)], axis=1)
    cntf = _make_sc_cnt(n_half, e2)(pidx, zrow, onerow)
    pre1, xr1 = _tc_linear_pair(x, wcat1, hid)
    agg1 = _make_sc_agg(n_half, hid, e2)(pre1, pidx, zrow)
    pre2, hr2 = _tc_combine_linear(agg1, cntf, xr1, b1, wcat2, lat, True, hid)
    agg2 = _make_sc_agg(n_half, hid, e2)(pre2, pidx, zrow)
    z, x_hat = _tc_final(agg2, cntf, hr2, b2, wdt, bdr, hid)
    lidx = jnp.stack([(prov + n_m).reshape(e // ECH, ECH),
                      memb.reshape(e // ECH, ECH)], axis=1)
    logits = _make_sc_edge_logits(e, hid, lat)(z, lidx)

    return (x_hat[:n_m], x_hat[n_m:], logits)
